# baseline (speedup 1.0000x reference)
import functools

import jax
import jax.numpy as jnp
from jax import lax
from jax.experimental import pallas as pl
from jax.experimental.pallas import tpu as pltpu
from jax.experimental.pallas import tpu_sc as plsc

N_F = 26
V = 100000
E = 32
B = 16384

NC = 2
NS = 16
NW = NC * NS
BPW = B // NW
CHUNK = 128
NCH = BPW // CHUNK


def _body(idx_hbm, tab_hbm, out_hbm, idx_v, rows_v, sem):
    wid = lax.axis_index("s") * NC + lax.axis_index("c")
    base = wid * BPW
    g0 = wid * NCH
    pltpu.sync_copy(idx_hbm.at[:, pl.ds(g0, NCH), :], idx_v)

    def field(f, carry):
        cps = []
        for c in range(NCH):
            cps.append(
                pltpu.async_copy(tab_hbm.at[idx_v.at[f, c]], rows_v.at[c], sem)
            )
        for cp in cps:
            cp.wait()
        for c in range(NCH):
            pltpu.sync_copy(
                rows_v.at[c],
                out_hbm.at[pl.ds(base + c * CHUNK, CHUNK), pl.ds(f * E, E)],
            )
        return carry

    lax.fori_loop(0, N_F, field, 0)


@functools.partial(
    pl.kernel,
    out_type=jax.ShapeDtypeStruct((B, N_F * E), jnp.float32),
    mesh=plsc.VectorSubcoreMesh(core_axis_name="c", subcore_axis_name="s"),
    compiler_params=pltpu.CompilerParams(use_tc_tiling_on_sc=False),
    scratch_types=[
        pltpu.VMEM((N_F, NCH, CHUNK), jnp.int32),
        pltpu.VMEM((NCH, CHUNK, E), jnp.float32),
        pltpu.SemaphoreType.DMA,
    ],
)
def _gather_kernel(idx_hbm, tab_hbm, out_hbm, idx_v, rows_v, sem):
    _body(idx_hbm, tab_hbm, out_hbm, idx_v, rows_v, sem)


def kernel(families, tables):
    fam = families.astype(jnp.int32)
    offs = (jnp.arange(N_F, dtype=jnp.int32) * V)[:, None]
    idx3 = (fam + offs).reshape(N_F, B // CHUNK, CHUNK)
    tab = tables.reshape(N_F * V, E)
    return _gather_kernel(idx3, tab)


# speedup vs baseline: 1.0138x; 1.0138x over previous
import functools

import jax
import jax.numpy as jnp
from jax import lax
from jax.experimental import pallas as pl
from jax.experimental.pallas import tpu as pltpu
from jax.experimental.pallas import tpu_sc as plsc

N_F = 26
V = 100000
E = 32
B = 16384

NC = 2
NS = 16
NW = NC * NS
BPW = B // NW
CHUNK = 128
NCH = BPW // CHUNK


def _body(idx_hbm, tab_hbm, out_hbm, idx_v, rows_v, gsem, wsem):
    wid = lax.axis_index("s") * NC + lax.axis_index("c")
    base = wid * BPW
    g0 = wid * NCH
    pltpu.sync_copy(idx_hbm.at[:, pl.ds(g0, NCH), :], idx_v)

    def g_start(f, p):
        for c in range(NCH):
            pltpu.make_async_copy(
                tab_hbm.at[idx_v.at[f, c]],
                rows_v.at[p, pl.ds(c * CHUNK, CHUNK), :],
                gsem,
            ).start()

    def g_wait(p):
        pltpu.make_async_copy(
            tab_hbm.at[pl.ds(0, BPW)], rows_v.at[p], gsem
        ).wait()

    def w_desc(f, p):
        return pltpu.make_async_copy(
            rows_v.at[p],
            out_hbm.at[pl.ds(base, BPW), pl.ds(f * E, E)],
            wsem,
        )

    g_start(0, 0)
    g_wait(0)
    g_start(1, 1)
    w_desc(0, 0).start()

    def field(f, carry):
        p = lax.rem(f, 2)
        q = 1 - p
        w_desc(f - 1, q).wait()
        g_start(f + 1, q)
        g_wait(p)
        w_desc(f, p).start()
        return carry

    lax.fori_loop(1, N_F - 1, field, 0)

    p = (N_F - 1) % 2
    q = 1 - p
    w_desc(N_F - 2, q).wait()
    g_wait(p)
    w_desc(N_F - 1, p).start()
    w_desc(N_F - 1, p).wait()


@functools.partial(
    pl.kernel,
    out_type=jax.ShapeDtypeStruct((B, N_F * E), jnp.float32),
    mesh=plsc.VectorSubcoreMesh(core_axis_name="c", subcore_axis_name="s"),
    compiler_params=pltpu.CompilerParams(use_tc_tiling_on_sc=False),
    scratch_types=[
        pltpu.VMEM((N_F, NCH, CHUNK), jnp.int32),
        pltpu.VMEM((2, BPW, E), jnp.float32),
        pltpu.SemaphoreType.DMA,
        pltpu.SemaphoreType.DMA,
    ],
)
def _gather_kernel(idx_hbm, tab_hbm, out_hbm, idx_v, rows_v, gsem, wsem):
    _body(idx_hbm, tab_hbm, out_hbm, idx_v, rows_v, gsem, wsem)


def kernel(families, tables):
    fam = families.astype(jnp.int32)
    offs = (jnp.arange(N_F, dtype=jnp.int32) * V)[:, None]
    idx3 = (fam + offs).reshape(N_F, B // CHUNK, CHUNK)
    tab = tables.reshape(N_F * V, E)
    return _gather_kernel(idx3, tab)


# speedup vs baseline: 1.0166x; 1.0028x over previous
import functools

import jax
import jax.numpy as jnp
from jax import lax
from jax.experimental import pallas as pl
from jax.experimental.pallas import tpu as pltpu
from jax.experimental.pallas import tpu_sc as plsc

N_F = 26
V = 100000
E = 32
B = 16384

NC = 2
NS = 16
NW = NC * NS
BPW = B // NW
CHUNK = 128
NCH = BPW // CHUNK


def _body(idx_hbm, tab_hbm, out_hbm, idx_v, rows_v, gsem, wsem):
    wid = lax.axis_index("s") * NC + lax.axis_index("c")
    base = wid * BPW
    pltpu.sync_copy(idx_hbm.at[:, pl.ds(base, BPW)], idx_v)

    def g_start(f, p):
        for c in range(NCH):
            pltpu.make_async_copy(
                tab_hbm.at[idx_v.at[f, pl.ds(c * CHUNK, CHUNK)]],
                rows_v.at[p, pl.ds(c * CHUNK, CHUNK), :],
                gsem,
            ).start()

    def g_wait(p):
        pltpu.make_async_copy(
            tab_hbm.at[pl.ds(0, BPW)], rows_v.at[p], gsem
        ).wait()

    def w_desc(f, p):
        return pltpu.make_async_copy(
            rows_v.at[p],
            out_hbm.at[pl.ds(base, BPW), pl.ds(f * E, E)],
            wsem,
        )

    g_start(0, 0)
    g_wait(0)
    g_start(1, 1)
    w_desc(0, 0).start()

    def field(f, carry):
        p = lax.rem(f, 2)
        q = 1 - p
        w_desc(f - 1, q).wait()
        g_start(f + 1, q)
        g_wait(p)
        w_desc(f, p).start()
        return carry

    lax.fori_loop(1, N_F - 1, field, 0)

    p = (N_F - 1) % 2
    q = 1 - p
    w_desc(N_F - 2, q).wait()
    g_wait(p)
    w_desc(N_F - 1, p).start()
    w_desc(N_F - 1, p).wait()


@functools.partial(
    pl.kernel,
    out_type=jax.ShapeDtypeStruct((B, N_F * E), jnp.float32),
    mesh=plsc.VectorSubcoreMesh(core_axis_name="c", subcore_axis_name="s"),
    compiler_params=pltpu.CompilerParams(use_tc_tiling_on_sc=False),
    scratch_types=[
        pltpu.VMEM((N_F, BPW), jnp.int32),
        pltpu.VMEM((2, BPW, E), jnp.float32),
        pltpu.SemaphoreType.DMA,
        pltpu.SemaphoreType.DMA,
    ],
)
def _gather_kernel(idx_hbm, tab_hbm, out_hbm, idx_v, rows_v, gsem, wsem):
    _body(idx_hbm, tab_hbm, out_hbm, idx_v, rows_v, gsem, wsem)


def kernel(families, tables):
    fam = families.astype(jnp.int32)
    offs = (jnp.arange(N_F, dtype=jnp.int32) * V)[:, None]
    idx2 = fam + offs
    tab = tables.reshape(N_F * V, E)
    return _gather_kernel(idx2, tab)
